# Initial kernel scaffold; baseline (speedup 1.0000x reference)
import jax
import jax.numpy as jnp
from jax.experimental import pallas as pl


def kernel(x, shift_state, token_ids, time_maa_k, time_maa_r, W_key, W_val, W_rec, Wk_e, Wv_e):
    raise NotImplementedError("write your pallas kernel here")


# speedup vs baseline: 1.0085x; 1.0085x over previous
import functools

import jax
import jax.numpy as jnp
from jax import lax
from jax.experimental import pallas as pl
from jax.experimental.pallas import tpu as pltpu
from jax.experimental.pallas import tpu_sc as plsc

HP = 5209
NE = 8
TM = 128
NT = 24
PTOT = TM * NT
TD = 256
FB = 896
FJ = 4
NC, NS = 2, 16
NW = NC * NS

_f32 = jnp.float32
_i32 = jnp.int32


def _shift_body(x_ref, xp_ref, mk_ref, mr_ref, xk_ref, xr_ref):
    x = x_ref[...]
    dx = xp_ref[...] - x
    xk_ref[...] = x + dx * mk_ref[...]
    xr_ref[...] = x + dx * mr_ref[...]


def _route_body(tid_ref, pos_ref, teid_ref):
    tid = tid_ref[...]
    eid = lax.rem(lax.rem(tid, HP), NE)

    c_i = lax.broadcasted_iota(_i32, (128, 128), 0)
    c_j = lax.broadcasted_iota(_i32, (128, 128), 1)
    m_tri = (c_i <= c_j).astype(_f32)
    r_i = lax.broadcasted_iota(_i32, (16, 16), 0)
    r_j = lax.broadcasted_iota(_i32, (16, 16), 1)
    l_tri = (r_j < r_i).astype(_f32)

    counts = []
    masks = []
    ranks = []
    for e in range(NE):
        m = eid == e
        mf = m.astype(_f32)
        within = jnp.dot(mf, m_tri, preferred_element_type=_f32)
        prevrows = jnp.dot(l_tri, mf, preferred_element_type=_f32)
        rowoff = jnp.sum(prevrows, axis=1, keepdims=True)
        rank = (within + rowoff).astype(_i32)
        masks.append(m)
        ranks.append(rank)
        counts.append(jnp.sum(m.astype(_i32)))

    starts = []
    s = jnp.int32(0)
    for e in range(NE):
        starts.append(s)
        s = s + ((counts[e] + (TM - 1)) // TM) * TM

    pos = jnp.zeros((16, 128), _i32)
    for e in range(NE):
        pos = jnp.where(masks[e], starts[e] + ranks[e] - 1, pos)
    pos_ref[...] = pos

    t_iota = lax.broadcasted_iota(_i32, (1, 128), 1) * TM
    te = jnp.zeros((1, 128), _i32)
    for e in range(1, NE):
        te = te + (t_iota >= starts[e]).astype(_i32)
    teid_ref[...] = te


def _moe_body(teid_ref, xs_ref, wk_ref, wv_ref, out_ref):
    del teid_ref
    h = lax.dot_general(xs_ref[...], wk_ref[0], (((1,), (1,)), ((), ())),
                        preferred_element_type=_f32)
    h = jnp.square(jnp.maximum(h, 0.0))
    out_ref[...] = lax.dot_general(h, wv_ref[0], (((1,), (1,)), ((), ())),
                                   preferred_element_type=_f32)


def _dense_body(xk_ref, xr_ref, wkey_ref, wval_ref, wrec_ref, dkv_ref,
                out_ref, acc_ref):
    j = pl.program_id(1)
    kp = lax.dot_general(xk_ref[...], wkey_ref[...], (((1,), (1,)), ((), ())),
                         preferred_element_type=_f32)
    kp = jnp.square(jnp.maximum(kp, 0.0))
    part = lax.dot_general(kp, wval_ref[...], (((1,), (1,)), ((), ())),
                           preferred_element_type=_f32)

    @pl.when(j == 0)
    def _():
        acc_ref[...] = part

    @pl.when(j > 0)
    def _():
        acc_ref[...] = acc_ref[...] + part

    @pl.when(j == FJ - 1)
    def _():
        r = jax.nn.sigmoid(
            lax.dot_general(xr_ref[...], wrec_ref[...], (((1,), (1,)), ((), ())),
                            preferred_element_type=_f32))
        out_ref[...] = r * (acc_ref[...] + dkv_ref[...])


def _make_sc_scatter(n, c, p):
    rp = n // NW
    mesh = plsc.VectorSubcoreMesh(core_axis_name="c", subcore_axis_name="s")

    @functools.partial(
        pl.kernel, mesh=mesh,
        out_type=jax.ShapeDtypeStruct((p, c), _f32),
        scratch_types=[pltpu.VMEM((rp,), _i32),
                       pltpu.VMEM((rp, c), _f32),
                       pltpu.SemaphoreType.DMA])
    def scat(src_hbm, pos_hbm, out_hbm, idx_v, rows_v, sem):
        wid = lax.axis_index("s") * NC + lax.axis_index("c")
        base = wid * rp
        pltpu.sync_copy(pos_hbm.at[pl.ds(base, rp)], idx_v)
        pltpu.sync_copy(src_hbm.at[pl.ds(base, rp)], rows_v)
        pltpu.async_copy(rows_v, out_hbm.at[idx_v], sem).wait()

    return scat


def _make_sc_gather(n, c, p):
    rp = n // NW
    mesh = plsc.VectorSubcoreMesh(core_axis_name="c", subcore_axis_name="s")

    @functools.partial(
        pl.kernel, mesh=mesh,
        out_type=jax.ShapeDtypeStruct((n, c), _f32),
        scratch_types=[pltpu.VMEM((rp,), _i32),
                       pltpu.VMEM((rp, c), _f32),
                       pltpu.SemaphoreType.DMA])
    def gath(src_hbm, pos_hbm, out_hbm, idx_v, rows_v, sem):
        wid = lax.axis_index("s") * NC + lax.axis_index("c")
        base = wid * rp
        pltpu.sync_copy(pos_hbm.at[pl.ds(base, rp)], idx_v)
        pltpu.async_copy(src_hbm.at[idx_v], rows_v, sem).wait()
        pltpu.sync_copy(rows_v, out_hbm.at[pl.ds(base, rp)])

    return gath


def kernel(x, shift_state, token_ids, time_maa_k, time_maa_r,
           W_key, W_val, W_rec, Wk_e, Wv_e):
    b, t, c = x.shape
    n = b * t
    fe = Wk_e.shape[1]
    f = W_key.shape[0]

    x2 = x.reshape(n, c)
    xprev = jnp.concatenate([shift_state[:, None, :], x[:, :-1]], axis=1)
    xp2 = xprev.reshape(n, c)
    mk = time_maa_k.reshape(1, c)
    mr = time_maa_r.reshape(1, c)

    nshift = n // TD
    xk, xr = pl.pallas_call(
        _shift_body,
        grid=(nshift,),
        in_specs=[pl.BlockSpec((TD, c), lambda i: (i, 0)),
                  pl.BlockSpec((TD, c), lambda i: (i, 0)),
                  pl.BlockSpec((1, c), lambda i: (0, 0)),
                  pl.BlockSpec((1, c), lambda i: (0, 0))],
        out_specs=[pl.BlockSpec((TD, c), lambda i: (i, 0)),
                   pl.BlockSpec((TD, c), lambda i: (i, 0))],
        out_shape=(jax.ShapeDtypeStruct((n, c), _f32),
                   jax.ShapeDtypeStruct((n, c), _f32)),
    )(x2, xp2, mk, mr)

    pos2d, teid2d = pl.pallas_call(
        _route_body,
        out_shape=(jax.ShapeDtypeStruct((16, 128), _i32),
                   jax.ShapeDtypeStruct((1, 128), _i32)),
    )(token_ids.reshape(16, 128))
    pos = pos2d.reshape(n)
    teid = teid2d.reshape(128)

    xk_sorted = _make_sc_scatter(n, c, PTOT)(xk, pos)

    moe_spec = pltpu.PrefetchScalarGridSpec(
        num_scalar_prefetch=1,
        grid=(NT,),
        in_specs=[pl.BlockSpec((TM, c), lambda i, te: (i, 0)),
                  pl.BlockSpec((1, fe, c), lambda i, te: (te[i], 0, 0)),
                  pl.BlockSpec((1, c, fe), lambda i, te: (te[i], 0, 0))],
        out_specs=pl.BlockSpec((TM, c), lambda i, te: (i, 0)),
    )
    dkv_sorted = pl.pallas_call(
        _moe_body, grid_spec=moe_spec,
        out_shape=jax.ShapeDtypeStruct((PTOT, c), _f32),
    )(teid, xk_sorted, Wk_e, Wv_e)

    dkv = _make_sc_gather(n, c, PTOT)(dkv_sorted, pos)

    out = pl.pallas_call(
        _dense_body,
        grid=(n // TD, FJ),
        in_specs=[pl.BlockSpec((TD, c), lambda i, j: (i, 0)),
                  pl.BlockSpec((TD, c), lambda i, j: (i, 0)),
                  pl.BlockSpec((FB, c), lambda i, j: (j, 0)),
                  pl.BlockSpec((c, FB), lambda i, j: (0, j)),
                  pl.BlockSpec((c, c), lambda i, j: (0, 0)),
                  pl.BlockSpec((TD, c), lambda i, j: (i, 0))],
        out_specs=pl.BlockSpec((TD, c), lambda i, j: (i, 0)),
        out_shape=jax.ShapeDtypeStruct((n, c), _f32),
        scratch_shapes=[pltpu.VMEM((TD, c), _f32)],
    )(xk, xr, W_key, W_val, W_rec, dkv)

    return out.reshape(b, t, c), x[:, -1]


# speedup vs baseline: 1.2588x; 1.2481x over previous
import functools

import jax
import jax.numpy as jnp
from jax import lax
from jax.experimental import pallas as pl
from jax.experimental.pallas import tpu as pltpu
from jax.experimental.pallas import tpu_sc as plsc

HP = 5209
NE = 8
TM = 128
NT = 24
PTOT = TM * NT
TD = 256
FB = 896
FJ = 4
NC, NS = 2, 16
NW = NC * NS

_f32 = jnp.float32
_i32 = jnp.int32


def _shift_body(x_ref, xp_ref, mk_ref, mr_ref, xk_ref, xr_ref):
    x = x_ref[...]
    dx = xp_ref[...] - x
    xk_ref[...] = x + dx * mk_ref[...]
    xr_ref[...] = x + dx * mr_ref[...]


def _route_body(tid_ref, pos_ref, teid_ref):
    tid = tid_ref[...]
    eid = lax.rem(lax.rem(tid, HP), NE)

    c_i = lax.broadcasted_iota(_i32, (128, 128), 0)
    c_j = lax.broadcasted_iota(_i32, (128, 128), 1)
    m_tri = (c_i <= c_j).astype(_f32)
    r_i = lax.broadcasted_iota(_i32, (16, 16), 0)
    r_j = lax.broadcasted_iota(_i32, (16, 16), 1)
    l_tri = (r_j < r_i).astype(_f32)

    counts = []
    masks = []
    ranks = []
    for e in range(NE):
        m = eid == e
        mf = m.astype(_f32)
        within = jnp.dot(mf, m_tri, preferred_element_type=_f32)
        prevrows = jnp.dot(l_tri, mf, preferred_element_type=_f32)
        rowoff = jnp.sum(prevrows, axis=1, keepdims=True)
        rank = (within + rowoff).astype(_i32)
        masks.append(m)
        ranks.append(rank)
        counts.append(jnp.sum(m.astype(_i32)))

    starts = []
    s = jnp.int32(0)
    for e in range(NE):
        starts.append(s)
        s = s + ((counts[e] + (TM - 1)) // TM) * TM

    pos = jnp.zeros((16, 128), _i32)
    for e in range(NE):
        pos = jnp.where(masks[e], starts[e] + ranks[e] - 1, pos)
    pos_ref[...] = pos

    t_iota = lax.broadcasted_iota(_i32, (1, 128), 1) * TM
    te = jnp.zeros((1, 128), _i32)
    for e in range(1, NE):
        te = te + (t_iota >= starts[e]).astype(_i32)
    teid_ref[...] = te


def _moe_body(teid_ref, xs_ref, wk_ref, wv_ref, out_ref):
    del teid_ref
    h = lax.dot_general(xs_ref[...], wk_ref[0], (((1,), (1,)), ((), ())),
                        preferred_element_type=_f32)
    h = jnp.square(jnp.maximum(h, 0.0))
    out_ref[...] = lax.dot_general(h, wv_ref[0], (((1,), (1,)), ((), ())),
                                   preferred_element_type=_f32)


def _dense_body(xk_ref, xr_ref, wkey_ref, wval_ref, wrec_ref, dkv_ref,
                out_ref):
    kp = lax.dot_general(xk_ref[...], wkey_ref[...], (((1,), (1,)), ((), ())),
                         preferred_element_type=_f32)
    kp = jnp.square(jnp.maximum(kp, 0.0))
    kv = lax.dot_general(kp, wval_ref[...], (((1,), (1,)), ((), ())),
                         preferred_element_type=_f32)
    r = jax.nn.sigmoid(
        lax.dot_general(xr_ref[...], wrec_ref[...], (((1,), (1,)), ((), ())),
                        preferred_element_type=_f32))
    out_ref[...] = r * (kv + dkv_ref[...])


def _make_sc_scatter(n, c, p):
    rp = n // NW
    mesh = plsc.VectorSubcoreMesh(core_axis_name="c", subcore_axis_name="s")

    @functools.partial(
        pl.kernel, mesh=mesh,
        out_type=jax.ShapeDtypeStruct((p, c), _f32),
        scratch_types=[pltpu.VMEM((rp,), _i32),
                       pltpu.VMEM((rp, c), _f32),
                       pltpu.SemaphoreType.DMA])
    def scat(src_hbm, pos_hbm, out_hbm, idx_v, rows_v, sem):
        wid = lax.axis_index("s") * NC + lax.axis_index("c")
        base = wid * rp
        pltpu.sync_copy(pos_hbm.at[pl.ds(base, rp)], idx_v)
        pltpu.sync_copy(src_hbm.at[pl.ds(base, rp)], rows_v)
        pltpu.async_copy(rows_v, out_hbm.at[idx_v], sem).wait()

    return scat


def _make_sc_gather(n, c, p):
    rp = n // NW
    mesh = plsc.VectorSubcoreMesh(core_axis_name="c", subcore_axis_name="s")

    @functools.partial(
        pl.kernel, mesh=mesh,
        out_type=jax.ShapeDtypeStruct((n, c), _f32),
        scratch_types=[pltpu.VMEM((rp,), _i32),
                       pltpu.VMEM((rp, c), _f32),
                       pltpu.SemaphoreType.DMA])
    def gath(src_hbm, pos_hbm, out_hbm, idx_v, rows_v, sem):
        wid = lax.axis_index("s") * NC + lax.axis_index("c")
        base = wid * rp
        pltpu.sync_copy(pos_hbm.at[pl.ds(base, rp)], idx_v)
        pltpu.async_copy(src_hbm.at[idx_v], rows_v, sem).wait()
        pltpu.sync_copy(rows_v, out_hbm.at[pl.ds(base, rp)])

    return gath


def kernel(x, shift_state, token_ids, time_maa_k, time_maa_r,
           W_key, W_val, W_rec, Wk_e, Wv_e):
    b, t, c = x.shape
    n = b * t
    fe = Wk_e.shape[1]
    f = W_key.shape[0]

    x2 = x.reshape(n, c)
    xprev = jnp.concatenate([shift_state[:, None, :], x[:, :-1]], axis=1)
    xp2 = xprev.reshape(n, c)
    mk = time_maa_k.reshape(1, c)
    mr = time_maa_r.reshape(1, c)

    nshift = n // TD
    xk, xr = pl.pallas_call(
        _shift_body,
        grid=(nshift,),
        in_specs=[pl.BlockSpec((TD, c), lambda i: (i, 0)),
                  pl.BlockSpec((TD, c), lambda i: (i, 0)),
                  pl.BlockSpec((1, c), lambda i: (0, 0)),
                  pl.BlockSpec((1, c), lambda i: (0, 0))],
        out_specs=[pl.BlockSpec((TD, c), lambda i: (i, 0)),
                   pl.BlockSpec((TD, c), lambda i: (i, 0))],
        out_shape=(jax.ShapeDtypeStruct((n, c), _f32),
                   jax.ShapeDtypeStruct((n, c), _f32)),
    )(x2, xp2, mk, mr)

    pos2d, teid2d = pl.pallas_call(
        _route_body,
        out_shape=(jax.ShapeDtypeStruct((16, 128), _i32),
                   jax.ShapeDtypeStruct((1, 128), _i32)),
    )(token_ids.reshape(16, 128))
    pos = pos2d.reshape(n)
    teid = teid2d.reshape(128)

    xk_sorted = _make_sc_scatter(n, c, PTOT)(xk, pos)

    moe_spec = pltpu.PrefetchScalarGridSpec(
        num_scalar_prefetch=1,
        grid=(NT,),
        in_specs=[pl.BlockSpec((TM, c), lambda i, te: (i, 0)),
                  pl.BlockSpec((1, fe, c), lambda i, te: (te[i], 0, 0)),
                  pl.BlockSpec((1, c, fe), lambda i, te: (te[i], 0, 0))],
        out_specs=pl.BlockSpec((TM, c), lambda i, te: (i, 0)),
    )
    dkv_sorted = pl.pallas_call(
        _moe_body, grid_spec=moe_spec,
        out_shape=jax.ShapeDtypeStruct((PTOT, c), _f32),
    )(teid, xk_sorted, Wk_e, Wv_e)

    dkv = _make_sc_gather(n, c, PTOT)(dkv_sorted, pos)

    out = pl.pallas_call(
        _dense_body,
        grid=(n // TD,),
        in_specs=[pl.BlockSpec((TD, c), lambda i: (i, 0)),
                  pl.BlockSpec((TD, c), lambda i: (i, 0)),
                  pl.BlockSpec((f, c), lambda i: (0, 0)),
                  pl.BlockSpec((c, f), lambda i: (0, 0)),
                  pl.BlockSpec((c, c), lambda i: (0, 0)),
                  pl.BlockSpec((TD, c), lambda i: (i, 0))],
        out_specs=pl.BlockSpec((TD, c), lambda i: (i, 0)),
        out_shape=jax.ShapeDtypeStruct((n, c), _f32),
    )(xk, xr, W_key, W_val, W_rec, dkv)

    return out.reshape(b, t, c), x[:, -1]
